# baseline (speedup 1.0000x reference)
import functools

import jax
import jax.numpy as jnp
from jax import lax
from jax.experimental import pallas as pl
from jax.experimental.pallas import tpu as pltpu
from jax.experimental.pallas import tpu_sc as plsc

B = 65536
D = 129
L = 16
NC = 2
NS = 16
NW = NC * NS
ROWS_PER_W = B // NW
R = 128
NCHUNK = ROWS_PER_W // R


def _body(in_hbm, out_hbm, in_v, out_v):
    cid = lax.axis_index("c")
    sid = lax.axis_index("s")
    wid = sid * NC + cid
    base = wid * ROWS_PER_W

    iota = lax.iota(jnp.int32, L)
    cols_ev = [2 * j0 + 2 * iota for j0 in range(0, 64, L)]
    cols_od = [2 * j0 + 1 + 2 * iota for j0 in range(0, 64, L)]
    col_last = jnp.full((L,), D - 1, jnp.int32)

    def chunk_body(c, carry):
        row0 = base + c * R
        pltpu.sync_copy(in_hbm.at[pl.ds(row0, R)], in_v)

        def row_body(r, carry2):
            rows = jnp.full((L,), r, jnp.int32)
            for k in range(4):
                j0 = k * L
                ev = plsc.load_gather(in_v, [rows, cols_ev[k]])
                out_v[r, pl.ds(j0, L)] = ev
                od = plsc.load_gather(in_v, [rows, cols_od[k]])
                out_v[r, pl.ds(64 + j0, L)] = od
            return carry2

        lax.fori_loop(0, R, row_body, 0)

        def tail_body(t, carry2):
            rows = t * L + iota
            val = plsc.load_gather(in_v, [rows, col_last])
            plsc.store_scatter(out_v, [rows, col_last], val)
            return carry2

        lax.fori_loop(0, R // L, tail_body, 0)

        pltpu.sync_copy(out_v, out_hbm.at[pl.ds(row0, R)])
        return carry

    lax.fori_loop(0, NCHUNK, chunk_body, 0)


@jax.jit
def kernel(tensor):
    mesh = plsc.VectorSubcoreMesh(core_axis_name="c", subcore_axis_name="s")
    f = functools.partial(
        pl.kernel,
        mesh=mesh,
        out_type=jax.ShapeDtypeStruct((B, D), jnp.float32),
        scratch_types=[
            pltpu.VMEM((R, D), jnp.float32),
            pltpu.VMEM((R, D), jnp.float32),
        ],
        compiler_params=pltpu.CompilerParams(
            use_tc_tiling_on_sc=False, needs_layout_passes=False
        ),
    )(_body)
    return f(tensor)


# speedup vs baseline: 1.1592x; 1.1592x over previous
import functools

import jax
import jax.numpy as jnp
from jax import lax
from jax.experimental import pallas as pl
from jax.experimental.pallas import tpu as pltpu
from jax.experimental.pallas import tpu_sc as plsc

B = 65536
D = 129
L = 16
NC = 2
NS = 16
NW = NC * NS
ROWS_PER_W = B // NW
R = 128
NCHUNK = ROWS_PER_W // R


def _body(in_hbm, out_hbm, in_v, out_v):
    cid = lax.axis_index("c")
    sid = lax.axis_index("s")
    wid = sid * NC + cid
    base = wid * ROWS_PER_W

    iota = lax.iota(jnp.int32, L)
    sc_cols = [
        (j0 + iota) // 2 + 64 * ((j0 + iota) % 2) for j0 in range(0, 2 * 64, L)
    ]
    col_last = jnp.full((L,), D - 1, jnp.int32)

    def chunk_body(c, carry):
        row0 = base + c * R
        pltpu.sync_copy(in_hbm.at[pl.ds(row0, R)], in_v)

        @plsc.parallel_loop(0, R, unroll=8)
        def row_body(r):
            rows = jnp.full((L,), r, jnp.int32)
            for k in range(8):
                x = in_v[r, pl.ds(k * L, L)]
                plsc.store_scatter(out_v, [rows, sc_cols[k]], x)

        @plsc.parallel_loop(0, R, step=L, unroll=2)
        def tail_body(t):
            rows = t + iota
            val = plsc.load_gather(in_v, [rows, col_last])
            plsc.store_scatter(out_v, [rows, col_last], val)

        pltpu.sync_copy(out_v, out_hbm.at[pl.ds(row0, R)])
        return carry

    lax.fori_loop(0, NCHUNK, chunk_body, 0)


@jax.jit
def kernel(tensor):
    mesh = plsc.VectorSubcoreMesh(core_axis_name="c", subcore_axis_name="s")
    f = functools.partial(
        pl.kernel,
        mesh=mesh,
        out_type=jax.ShapeDtypeStruct((B, D), jnp.float32),
        scratch_types=[
            pltpu.VMEM((R, D), jnp.float32),
            pltpu.VMEM((R, D), jnp.float32),
        ],
        compiler_params=pltpu.CompilerParams(
            use_tc_tiling_on_sc=False, needs_layout_passes=False
        ),
    )(_body)
    return f(tensor)


# speedup vs baseline: 2.3201x; 2.0015x over previous
import functools

import jax
import jax.numpy as jnp
from jax import lax
from jax.experimental import pallas as pl
from jax.experimental.pallas import tpu as pltpu
from jax.experimental.pallas import tpu_sc as plsc

B = 65536
D = 129
L = 16
NC = 2
NS = 16
NW = NC * NS
ROWS_PER_W = B // NW
R = 128
NCHUNK = ROWS_PER_W // R


def _body(in_hbm, out_hbm, in_v, out_v):
    cid = lax.axis_index("c")
    sid = lax.axis_index("s")
    wid = sid * NC + cid
    base = wid * ROWS_PER_W

    iota = lax.iota(jnp.int32, L)
    sc_cols = [
        (j0 + iota) // 2 + 64 * ((j0 + iota) % 2) for j0 in range(0, 2 * 64, L)
    ]
    col_last = jnp.full((L,), D - 1, jnp.int32)

    def chunk_body(c, carry):
        row0 = base + c * R
        pltpu.sync_copy(in_hbm.at[pl.ds(row0, R)], in_v)

        @plsc.parallel_loop(0, R, unroll=8)
        def row_body(r):
            rows = jnp.full((L,), r, jnp.int32)
            for k in range(8):
                x = in_v[r, pl.ds(k * L, L)]
                plsc.store_scatter(out_v, [rows, sc_cols[k]], x)

        @plsc.parallel_loop(0, R, step=L, unroll=2)
        def tail_body(t):
            rows = t + iota
            val = plsc.load_gather(in_v, [rows, col_last])
            plsc.store_scatter(out_v, [rows, col_last], val)

        pltpu.sync_copy(out_v, out_hbm.at[pl.ds(row0, R)])
        return carry

    lax.fori_loop(0, NCHUNK, chunk_body, 0)


@jax.jit
def kernel(tensor):
    mesh = plsc.VectorSubcoreMesh(core_axis_name="c", subcore_axis_name="s")
    f = functools.partial(
        pl.kernel,
        mesh=mesh,
        out_type=jax.ShapeDtypeStruct((B, D), jnp.float32),
        scratch_types=[
            pltpu.VMEM((R, D), jnp.float32),
            pltpu.VMEM((R, D), jnp.float32),
        ],
        compiler_params=pltpu.CompilerParams(
            use_tc_tiling_on_sc=True, needs_layout_passes=False
        ),
    )(_body)
    return f(tensor)


# speedup vs baseline: 2.6291x; 1.1332x over previous
import functools

import jax
import jax.numpy as jnp
from jax import lax
from jax.experimental import pallas as pl
from jax.experimental.pallas import tpu as pltpu
from jax.experimental.pallas import tpu_sc as plsc

B = 65536
D = 129
L = 16
NC = 2
NS = 16
NW = NC * NS
ROWS_PER_W = B // NW
R = 128
NCHUNK = ROWS_PER_W // R
NPAIR = NCHUNK // 2


def _body(in_hbm, out_hbm, in0, in1, out0, out1, si0, si1, so0, so1):
    cid = lax.axis_index("c")
    sid = lax.axis_index("s")
    wid = sid * NC + cid
    base = wid * ROWS_PER_W

    in_bufs = (in0, in1)
    out_bufs = (out0, out1)
    isems = (si0, si1)
    osems = (so0, so1)

    iota = lax.iota(jnp.int32, L)
    srcs = [32 * k + 2 * iota for k in range(4)]
    srcs += [32 * k + 2 * iota + 1 for k in range(4)]
    col_last = jnp.full((L,), D - 1, jnp.int32)

    pltpu.async_copy(in_hbm.at[pl.ds(base, R)], in0, si0)
    pltpu.async_copy(in_hbm.at[pl.ds(base + R, R)], in1, si1)

    def pair_body(t, carry):
        for b in range(2):
            c = 2 * t + b
            row0 = base + c * R
            iv, ov = in_bufs[b], out_bufs[b]
            isem, osem = isems[b], osems[b]

            pltpu.make_async_copy(in_hbm.at[pl.ds(row0, R)], iv, isem).wait()

            @pl.when(t > 0)
            def _():
                pltpu.make_async_copy(
                    ov, out_hbm.at[pl.ds(row0, R)], osem
                ).wait()

            @plsc.parallel_loop(0, R, unroll=4)
            def row_body(r):
                rfull = jnp.full((L,), r, jnp.int32)
                for k in range(8):
                    ov[r, pl.ds(k * L, L)] = plsc.load_gather(
                        iv, [rfull, srcs[k]]
                    )

            @plsc.parallel_loop(0, R, step=L, unroll=2)
            def tail_body(tt):
                rows = tt + iota
                val = plsc.load_gather(iv, [rows, col_last])
                plsc.store_scatter(ov, [rows, col_last], val)

            @pl.when(t < NPAIR - 1)
            def _():
                pltpu.async_copy(
                    in_hbm.at[pl.ds(row0 + 2 * R, R)], iv, isem
                )

            pltpu.async_copy(ov, out_hbm.at[pl.ds(row0, R)], osem)
        return carry

    lax.fori_loop(0, NPAIR, pair_body, 0)

    pltpu.make_async_copy(out0, out_hbm.at[pl.ds(base, R)], so0).wait()
    pltpu.make_async_copy(out1, out_hbm.at[pl.ds(base + R, R)], so1).wait()


@jax.jit
def kernel(tensor):
    mesh = plsc.VectorSubcoreMesh(core_axis_name="c", subcore_axis_name="s")
    f = functools.partial(
        pl.kernel,
        mesh=mesh,
        out_type=jax.ShapeDtypeStruct((B, D), jnp.float32),
        scratch_types=[
            pltpu.VMEM((R, D), jnp.float32),
            pltpu.VMEM((R, D), jnp.float32),
            pltpu.VMEM((R, D), jnp.float32),
            pltpu.VMEM((R, D), jnp.float32),
            pltpu.SemaphoreType.DMA,
            pltpu.SemaphoreType.DMA,
            pltpu.SemaphoreType.DMA,
            pltpu.SemaphoreType.DMA,
        ],
        compiler_params=pltpu.CompilerParams(
            use_tc_tiling_on_sc=True, needs_layout_passes=False
        ),
    )(_body)
    return f(tensor)
